# baseline (speedup 1.0000x reference)
import functools

import jax
import jax.numpy as jnp
from jax import lax
from jax.experimental import pallas as pl
from jax.experimental.pallas import tpu as pltpu
from jax.experimental.pallas import tpu_sc as plsc

_NUM_POS = 2304
_HIDDEN = 1152
_MERGE = 2
_GRID = 48
_F = 4
_H = 128
_W = 128
_ROWS = _H * _W
_NW = 32
_RPW = _ROWS // _NW
_B = 16
_NCH = _RPW // _B
_NG = _HIDDEN // 32


def _linspace(stop, num, num_static):
    div = (jnp.asarray(num) - 1).astype(jnp.float32)
    delta = jnp.float32(stop) / div
    body = lax.iota(jnp.float32, num_static - 1) * delta
    return jnp.concatenate([body, jnp.full((1,), stop, dtype=jnp.float32)])


def _sc_body(table_hbm, i0_hbm, i1_hbm, i2_hbm, i3_hbm,
             wall_hbm, c_hbm, out_hbm,
             i0v, i1v, i2v, i3v, wallv, cv,
             r0, r1, r2, r3, ov, sem, osem):
    wid = lax.axis_index("s") * 2 + lax.axis_index("c")
    base = wid * _RPW
    pltpu.sync_copy(i0_hbm.at[pl.ds(base, _RPW)], i0v)
    pltpu.sync_copy(i1_hbm.at[pl.ds(base, _RPW)], i1v)
    pltpu.sync_copy(i2_hbm.at[pl.ds(base, _RPW)], i2v)
    pltpu.sync_copy(i3_hbm.at[pl.ds(base, _RPW)], i3v)
    pltpu.sync_copy(wall_hbm.at[pl.ds(base * 4, _RPW * 4)],
                    wallv.at[pl.ds(0, _RPW * 4)])
    pltpu.sync_copy(c_hbm, cv)

    def chunk_body(ch, _):
        off = ch * _B
        iv0 = i0v[pl.ds(off, _B)]
        iv1 = i1v[pl.ds(off, _B)]
        iv2 = i2v[pl.ds(off, _B)]
        iv3 = i3v[pl.ds(off, _B)]
        handles = []
        for p in range(_B):
            o0 = pl.multiple_of(iv0[p], 8)
            o1 = pl.multiple_of(iv1[p], 8)
            o2 = pl.multiple_of(iv2[p], 8)
            o3 = pl.multiple_of(iv3[p], 8)
            handles.append(pltpu.async_copy(
                table_hbm.at[pl.ds(o0, _HIDDEN)],
                r0.at[pl.ds(p * _HIDDEN, _HIDDEN)], sem))
            handles.append(pltpu.async_copy(
                table_hbm.at[pl.ds(o1, _HIDDEN)],
                r1.at[pl.ds(p * _HIDDEN, _HIDDEN)], sem))
            handles.append(pltpu.async_copy(
                table_hbm.at[pl.ds(o2, _HIDDEN)],
                r2.at[pl.ds(p * _HIDDEN, _HIDDEN)], sem))
            handles.append(pltpu.async_copy(
                table_hbm.at[pl.ds(o3, _HIDDEN)],
                r3.at[pl.ds(p * _HIDDEN, _HIDDEN)], sem))
        for h in handles:
            h.wait()
        cvec = cv[...]

        def pos_body(p, _):
            wq = wallv[pl.ds((off + p) * 4, 16)]
            a0 = wq[0]
            a1 = wq[1]
            a2 = wq[2]
            a3 = wq[3]

            pb = p * _HIDDEN
            for g in range(_NG):
                ce = (a0 * r0[pl.ds(pb + g * 32, 16)]
                      + a1 * r1[pl.ds(pb + g * 32, 16)]
                      + a2 * r2[pl.ds(pb + g * 32, 16)]
                      + a3 * r3[pl.ds(pb + g * 32, 16)] + cvec)
                co = (a0 * r0[pl.ds(pb + g * 32 + 16, 16)]
                      + a1 * r1[pl.ds(pb + g * 32 + 16, 16)]
                      + a2 * r2[pl.ds(pb + g * 32 + 16, 16)]
                      + a3 * r3[pl.ds(pb + g * 32 + 16, 16)] + cvec)
                ov[p, pl.ds(g * 32, 32)] = plsc.pack(
                    ce, co, format=plsc.PackFormat.INTERLEAVED)
            return ()

        lax.fori_loop(0, _B, pos_body, ())
        s0 = pltpu.async_copy(ov, out_hbm.at[pl.ds(base + off, _B)], osem)
        s1 = pltpu.async_copy(ov, out_hbm.at[pl.ds(_ROWS + base + off, _B)], osem)
        s2 = pltpu.async_copy(ov, out_hbm.at[pl.ds(2 * _ROWS + base + off, _B)], osem)
        s3 = pltpu.async_copy(ov, out_hbm.at[pl.ds(3 * _ROWS + base + off, _B)], osem)
        s0.wait()
        s1.wait()
        s2.wait()
        s3.wait()
        return ()

    lax.fori_loop(0, _NCH, chunk_body, ())


@functools.partial(
    pl.kernel,
    out_type=jax.ShapeDtypeStruct((_F * _ROWS, _HIDDEN), jnp.bfloat16),
    mesh=plsc.VectorSubcoreMesh(core_axis_name="c", subcore_axis_name="s"),
    compiler_params=pltpu.CompilerParams(needs_layout_passes=False),
    scratch_types=[
        pltpu.VMEM((_RPW,), jnp.int32),
        pltpu.VMEM((_RPW,), jnp.int32),
        pltpu.VMEM((_RPW,), jnp.int32),
        pltpu.VMEM((_RPW,), jnp.int32),
        pltpu.VMEM((_RPW * 4 + 16,), jnp.float32),
        pltpu.VMEM((16,), jnp.float32),
        pltpu.VMEM((_B * _HIDDEN,), jnp.float32),
        pltpu.VMEM((_B * _HIDDEN,), jnp.float32),
        pltpu.VMEM((_B * _HIDDEN,), jnp.float32),
        pltpu.VMEM((_B * _HIDDEN,), jnp.float32),
        pltpu.VMEM((_B, _HIDDEN), jnp.bfloat16),
        pltpu.SemaphoreType.DMA,
        pltpu.SemaphoreType.DMA,
    ],
)
def _sc_interp(table_hbm, i0_hbm, i1_hbm, i2_hbm, i3_hbm,
               wall_hbm, c_hbm, out_hbm,
               i0v, i1v, i2v, i3v, wallv, cv,
               r0, r1, r2, r3, ov, sem, osem):
    _sc_body(table_hbm, i0_hbm, i1_hbm, i2_hbm, i3_hbm,
             wall_hbm, c_hbm, out_hbm,
             i0v, i1v, i2v, i3v, wallv, cv,
             r0, r1, r2, r3, ov, sem, osem)


def kernel(num_frames, height, width, pos_embed):
    h_idxs = _linspace(_GRID - 1, height, _H)
    w_idxs = _linspace(_GRID - 1, width, _W)
    hf = jnp.floor(h_idxs).astype(jnp.int32)
    wf = jnp.floor(w_idxs).astype(jnp.int32)
    hc = jnp.minimum(hf + 1, _GRID - 1)
    wc = jnp.minimum(wf + 1, _GRID - 1)
    dh = h_idxs - hf
    dw = w_idxs - wf

    r = jnp.arange(_ROWS)
    m = r // (_MERGE * _MERGE * (_W // _MERGE))
    rem = r % (_MERGE * _MERGE * (_W // _MERGE))
    n = rem // (_MERGE * _MERGE)
    ij = rem % (_MERGE * _MERGE)
    i = ij // _MERGE
    j = ij % _MERGE
    h = _MERGE * m + i
    w = _MERGE * n + j

    hfr = hf[h]
    hcr = hc[h]
    wfr = wf[w]
    wcr = wc[w]
    dhr = dh[h]
    dwr = dw[w]
    i0 = (hfr * _GRID + wfr) * _HIDDEN
    i1 = (hfr * _GRID + wcr) * _HIDDEN
    i2 = (hcr * _GRID + wfr) * _HIDDEN
    i3 = (hcr * _GRID + wcr) * _HIDDEN
    w0 = (1 - dhr) * (1 - dwr)
    w1 = (1 - dhr) * dwr
    w2 = dhr * (1 - dwr)
    w3 = dhr * dwr
    wall = jnp.stack([w0, w1, w2, w3], axis=1).reshape(-1)

    g = jnp.arange(_HIDDEN)
    grp = g // 32
    lane = g % 32
    colperm = grp * 32 + jnp.where(lane < 16, 2 * lane, 2 * (lane - 16) + 1)
    table_p = pos_embed[:, colperm].reshape(-1)

    cvec = jnp.full((16,), (jnp.asarray(num_frames) - _F), dtype=jnp.float32)

    return _sc_interp(table_p, i0, i1, i2, i3, wall, cvec)


# speedup vs baseline: 1.6803x; 1.6803x over previous
import functools

import jax
import jax.numpy as jnp
from jax import lax
from jax.experimental import pallas as pl
from jax.experimental.pallas import tpu as pltpu
from jax.experimental.pallas import tpu_sc as plsc

_NUM_POS = 2304
_HIDDEN = 1152
_MERGE = 2
_GRID = 48
_F = 4
_H = 128
_W = 128
_ROWS = _H * _W
_NW = 32
_RPW = _ROWS // _NW
_B = 16
_NCH = _RPW // _B
_NG = _HIDDEN // 32


def _linspace(stop, num, num_static):
    div = (jnp.asarray(num) - 1).astype(jnp.float32)
    delta = jnp.float32(stop) / div
    body = lax.iota(jnp.float32, num_static - 1) * delta
    return jnp.concatenate([body, jnp.full((1,), stop, dtype=jnp.float32)])


def _sc_body(table_hbm, i0_hbm, i1_hbm, i2_hbm, i3_hbm,
             wall_hbm, c_hbm, out_hbm,
             i0v, i1v, i2v, i3v, wallv, cv,
             r0, r1, r2, r3, ov, sem, osem):
    wid = lax.axis_index("s") * 2 + lax.axis_index("c")
    base = wid * _RPW
    pltpu.sync_copy(i0_hbm.at[pl.ds(base, _RPW)], i0v)
    pltpu.sync_copy(i1_hbm.at[pl.ds(base, _RPW)], i1v)
    pltpu.sync_copy(i2_hbm.at[pl.ds(base, _RPW)], i2v)
    pltpu.sync_copy(i3_hbm.at[pl.ds(base, _RPW)], i3v)
    pltpu.sync_copy(wall_hbm.at[pl.ds(base * 4, _RPW * 4)],
                    wallv.at[pl.ds(0, _RPW * 4)])
    pltpu.sync_copy(c_hbm, cv)

    def chunk_body(ch, _):
        off = ch * _B
        iv0 = i0v[pl.ds(off, _B)]
        iv1 = i1v[pl.ds(off, _B)]
        iv2 = i2v[pl.ds(off, _B)]
        iv3 = i3v[pl.ds(off, _B)]
        handles = []
        for p in range(_B):
            o0 = pl.multiple_of(iv0[p], 8)
            o1 = pl.multiple_of(iv1[p], 8)
            o2 = pl.multiple_of(iv2[p], 8)
            o3 = pl.multiple_of(iv3[p], 8)
            handles.append(pltpu.async_copy(
                table_hbm.at[pl.ds(o0, _HIDDEN)],
                r0.at[pl.ds(p * _HIDDEN, _HIDDEN)], sem))
            handles.append(pltpu.async_copy(
                table_hbm.at[pl.ds(o1, _HIDDEN)],
                r1.at[pl.ds(p * _HIDDEN, _HIDDEN)], sem))
            handles.append(pltpu.async_copy(
                table_hbm.at[pl.ds(o2, _HIDDEN)],
                r2.at[pl.ds(p * _HIDDEN, _HIDDEN)], sem))
            handles.append(pltpu.async_copy(
                table_hbm.at[pl.ds(o3, _HIDDEN)],
                r3.at[pl.ds(p * _HIDDEN, _HIDDEN)], sem))
        for h in handles:
            h.wait()
        cvec = cv[...]

        def pos_body(p, _):
            wq = wallv[pl.ds((off + p) * 4, 16)]
            a0 = wq[0]
            a1 = wq[1]
            a2 = wq[2]
            a3 = wq[3]

            pb = p * _HIDDEN
            for g in range(_NG):
                ce = (a0 * r0[pl.ds(pb + g * 32, 16)]
                      + a1 * r1[pl.ds(pb + g * 32, 16)]
                      + a2 * r2[pl.ds(pb + g * 32, 16)]
                      + a3 * r3[pl.ds(pb + g * 32, 16)] + cvec)
                co = (a0 * r0[pl.ds(pb + g * 32 + 16, 16)]
                      + a1 * r1[pl.ds(pb + g * 32 + 16, 16)]
                      + a2 * r2[pl.ds(pb + g * 32 + 16, 16)]
                      + a3 * r3[pl.ds(pb + g * 32 + 16, 16)] + cvec)
                ov[p, pl.ds(g * 32, 32)] = plsc.pack(
                    ce, co, format=plsc.PackFormat.INTERLEAVED)
            return ()

        lax.fori_loop(0, _B, pos_body, ())
        s0 = pltpu.async_copy(ov, out_hbm.at[pl.ds(base + off, _B)], osem)
        s1 = pltpu.async_copy(ov, out_hbm.at[pl.ds(_ROWS + base + off, _B)], osem)
        s2 = pltpu.async_copy(ov, out_hbm.at[pl.ds(2 * _ROWS + base + off, _B)], osem)
        s3 = pltpu.async_copy(ov, out_hbm.at[pl.ds(3 * _ROWS + base + off, _B)], osem)
        s0.wait()
        s1.wait()
        s2.wait()
        s3.wait()
        return ()

    lax.fori_loop(0, _NCH, chunk_body, ())


@functools.partial(
    pl.kernel,
    out_type=jax.ShapeDtypeStruct((_F * _ROWS, _HIDDEN), jnp.bfloat16),
    mesh=plsc.VectorSubcoreMesh(core_axis_name="c", subcore_axis_name="s"),
    compiler_params=pltpu.CompilerParams(needs_layout_passes=False),
    scratch_types=[
        pltpu.VMEM((_RPW,), jnp.int32),
        pltpu.VMEM((_RPW,), jnp.int32),
        pltpu.VMEM((_RPW,), jnp.int32),
        pltpu.VMEM((_RPW,), jnp.int32),
        pltpu.VMEM((_RPW * 4 + 16,), jnp.float32),
        pltpu.VMEM((16,), jnp.float32),
        pltpu.VMEM((_B * _HIDDEN,), jnp.float32),
        pltpu.VMEM((_B * _HIDDEN,), jnp.float32),
        pltpu.VMEM((_B * _HIDDEN,), jnp.float32),
        pltpu.VMEM((_B * _HIDDEN,), jnp.float32),
        pltpu.VMEM((_B, _HIDDEN), jnp.bfloat16),
        pltpu.SemaphoreType.DMA,
        pltpu.SemaphoreType.DMA,
    ],
)
def _sc_interp(table_hbm, i0_hbm, i1_hbm, i2_hbm, i3_hbm,
               wall_hbm, c_hbm, out_hbm,
               i0v, i1v, i2v, i3v, wallv, cv,
               r0, r1, r2, r3, ov, sem, osem):
    _sc_body(table_hbm, i0_hbm, i1_hbm, i2_hbm, i3_hbm,
             wall_hbm, c_hbm, out_hbm,
             i0v, i1v, i2v, i3v, wallv, cv,
             r0, r1, r2, r3, ov, sem, osem)


def kernel(num_frames, height, width, pos_embed):
    h_idxs = _linspace(_GRID - 1, height, _H)
    w_idxs = _linspace(_GRID - 1, width, _W)
    hf = jnp.floor(h_idxs).astype(jnp.int32)
    wf = jnp.floor(w_idxs).astype(jnp.int32)
    hc = jnp.minimum(hf + 1, _GRID - 1)
    wc = jnp.minimum(wf + 1, _GRID - 1)
    dh = h_idxs - hf
    dw = w_idxs - wf

    hf_mi = hf.reshape(_H // _MERGE, _MERGE)[:, None, :, None]
    hc_mi = hc.reshape(_H // _MERGE, _MERGE)[:, None, :, None]
    dh_mi = dh.reshape(_H // _MERGE, _MERGE)[:, None, :, None]
    wf_nj = wf.reshape(_W // _MERGE, _MERGE)[None, :, None, :]
    wc_nj = wc.reshape(_W // _MERGE, _MERGE)[None, :, None, :]
    dw_nj = dw.reshape(_W // _MERGE, _MERGE)[None, :, None, :]
    i0 = ((hf_mi * _GRID + wf_nj) * _HIDDEN).reshape(-1)
    i1 = ((hf_mi * _GRID + wc_nj) * _HIDDEN).reshape(-1)
    i2 = ((hc_mi * _GRID + wf_nj) * _HIDDEN).reshape(-1)
    i3 = ((hc_mi * _GRID + wc_nj) * _HIDDEN).reshape(-1)
    w0 = (1 - dh_mi) * (1 - dw_nj)
    w1 = (1 - dh_mi) * dw_nj
    w2 = dh_mi * (1 - dw_nj)
    w3 = dh_mi * dw_nj
    wall = jnp.stack(
        [jnp.broadcast_to(x, (_H // _MERGE, _W // _MERGE, _MERGE, _MERGE))
         for x in (w0, w1, w2, w3)], axis=-1).reshape(-1)

    table_p = (pos_embed.reshape(_NUM_POS, _NG, 16, 2)
               .transpose(0, 1, 3, 2).reshape(-1))

    cvec = jnp.full((16,), (jnp.asarray(num_frames) - _F), dtype=jnp.float32)

    return _sc_interp(table_p, i0, i1, i2, i3, wall, cvec)


# speedup vs baseline: 2.2560x; 1.3427x over previous
import functools

import jax
import jax.numpy as jnp
from jax import lax
from jax.experimental import pallas as pl
from jax.experimental.pallas import tpu as pltpu
from jax.experimental.pallas import tpu_sc as plsc

_NUM_POS = 2304
_HIDDEN = 1152
_MERGE = 2
_GRID = 48
_F = 4
_H = 128
_W = 128
_ROWS = _H * _W
_NW = 32
_RPW = _ROWS // _NW
_B = 16
_NCH = _RPW // _B
_NG = _HIDDEN // 32


def _linspace(stop, num, num_static):
    div = (jnp.asarray(num) - 1).astype(jnp.float32)
    delta = jnp.float32(stop) / div
    body = lax.iota(jnp.float32, num_static - 1) * delta
    return jnp.concatenate([body, jnp.full((1,), stop, dtype=jnp.float32)])


def _sc_body(table_hbm, i0_hbm, i1_hbm, i2_hbm, i3_hbm,
             wall_hbm, c_hbm, out_hbm,
             i0v, i1v, i2v, i3v, wallv, cv,
             r0, r1, r2, r3, ov, sem, osem):
    wid = lax.axis_index("s") * 2 + lax.axis_index("c")
    base = wid * _RPW
    pltpu.sync_copy(i0_hbm.at[pl.ds(base, _RPW)], i0v)
    pltpu.sync_copy(i1_hbm.at[pl.ds(base, _RPW)], i1v)
    pltpu.sync_copy(i2_hbm.at[pl.ds(base, _RPW)], i2v)
    pltpu.sync_copy(i3_hbm.at[pl.ds(base, _RPW)], i3v)
    pltpu.sync_copy(wall_hbm.at[pl.ds(base * 4, _RPW * 4)],
                    wallv.at[pl.ds(0, _RPW * 4)])
    pltpu.sync_copy(c_hbm, cv)

    def chunk_body(ch, _):
        off = ch * _B
        g0 = pltpu.async_copy(table_hbm.at[i0v[pl.ds(off, _B)]], r0, sem)
        g1 = pltpu.async_copy(table_hbm.at[i1v[pl.ds(off, _B)]], r1, sem)
        g2 = pltpu.async_copy(table_hbm.at[i2v[pl.ds(off, _B)]], r2, sem)
        g3 = pltpu.async_copy(table_hbm.at[i3v[pl.ds(off, _B)]], r3, sem)
        g0.wait()
        g1.wait()
        g2.wait()
        g3.wait()
        cvec = cv[...]
        lane = lax.iota(jnp.int32, 16)
        sh_e = (2 * lane) & 15
        sh_o = sh_e + 1
        lo_half = lane < 8

        def pos_body(p, _):
            wq = wallv[pl.ds((off + p) * 4, 16)]
            a0 = wq[0]
            a1 = wq[1]
            a2 = wq[2]
            a3 = wq[3]

            for g in range(_NG):
                acc0 = (a0 * r0[p, pl.ds(g * 32, 16)]
                        + a1 * r1[p, pl.ds(g * 32, 16)]
                        + a2 * r2[p, pl.ds(g * 32, 16)]
                        + a3 * r3[p, pl.ds(g * 32, 16)] + cvec)
                acc1 = (a0 * r0[p, pl.ds(g * 32 + 16, 16)]
                        + a1 * r1[p, pl.ds(g * 32 + 16, 16)]
                        + a2 * r2[p, pl.ds(g * 32 + 16, 16)]
                        + a3 * r3[p, pl.ds(g * 32 + 16, 16)] + cvec)
                ce = jnp.where(lo_half, jnp.take(acc0, sh_e), jnp.take(acc1, sh_e))
                co = jnp.where(lo_half, jnp.take(acc0, sh_o), jnp.take(acc1, sh_o))
                ov[p, pl.ds(g * 32, 32)] = plsc.pack(
                    ce, co, format=plsc.PackFormat.INTERLEAVED)
            return ()

        lax.fori_loop(0, _B, pos_body, ())
        s0 = pltpu.async_copy(ov, out_hbm.at[pl.ds(base + off, _B)], osem)
        s1 = pltpu.async_copy(ov, out_hbm.at[pl.ds(_ROWS + base + off, _B)], osem)
        s2 = pltpu.async_copy(ov, out_hbm.at[pl.ds(2 * _ROWS + base + off, _B)], osem)
        s3 = pltpu.async_copy(ov, out_hbm.at[pl.ds(3 * _ROWS + base + off, _B)], osem)
        s0.wait()
        s1.wait()
        s2.wait()
        s3.wait()
        return ()

    lax.fori_loop(0, _NCH, chunk_body, ())


@functools.partial(
    pl.kernel,
    out_type=jax.ShapeDtypeStruct((_F * _ROWS, _HIDDEN), jnp.bfloat16),
    mesh=plsc.VectorSubcoreMesh(core_axis_name="c", subcore_axis_name="s"),
    compiler_params=pltpu.CompilerParams(needs_layout_passes=False),
    scratch_types=[
        pltpu.VMEM((_RPW,), jnp.int32),
        pltpu.VMEM((_RPW,), jnp.int32),
        pltpu.VMEM((_RPW,), jnp.int32),
        pltpu.VMEM((_RPW,), jnp.int32),
        pltpu.VMEM((_RPW * 4 + 16,), jnp.float32),
        pltpu.VMEM((16,), jnp.float32),
        pltpu.VMEM((_B, _HIDDEN), jnp.float32),
        pltpu.VMEM((_B, _HIDDEN), jnp.float32),
        pltpu.VMEM((_B, _HIDDEN), jnp.float32),
        pltpu.VMEM((_B, _HIDDEN), jnp.float32),
        pltpu.VMEM((_B, _HIDDEN), jnp.bfloat16),
        pltpu.SemaphoreType.DMA,
        pltpu.SemaphoreType.DMA,
    ],
)
def _sc_interp(table_hbm, i0_hbm, i1_hbm, i2_hbm, i3_hbm,
               wall_hbm, c_hbm, out_hbm,
               i0v, i1v, i2v, i3v, wallv, cv,
               r0, r1, r2, r3, ov, sem, osem):
    _sc_body(table_hbm, i0_hbm, i1_hbm, i2_hbm, i3_hbm,
             wall_hbm, c_hbm, out_hbm,
             i0v, i1v, i2v, i3v, wallv, cv,
             r0, r1, r2, r3, ov, sem, osem)


def kernel(num_frames, height, width, pos_embed):
    h_idxs = _linspace(_GRID - 1, height, _H)
    w_idxs = _linspace(_GRID - 1, width, _W)
    hf = jnp.floor(h_idxs).astype(jnp.int32)
    wf = jnp.floor(w_idxs).astype(jnp.int32)
    hc = jnp.minimum(hf + 1, _GRID - 1)
    wc = jnp.minimum(wf + 1, _GRID - 1)
    dh = h_idxs - hf
    dw = w_idxs - wf

    hf_mi = hf.reshape(_H // _MERGE, _MERGE)[:, None, :, None]
    hc_mi = hc.reshape(_H // _MERGE, _MERGE)[:, None, :, None]
    dh_mi = dh.reshape(_H // _MERGE, _MERGE)[:, None, :, None]
    wf_nj = wf.reshape(_W // _MERGE, _MERGE)[None, :, None, :]
    wc_nj = wc.reshape(_W // _MERGE, _MERGE)[None, :, None, :]
    dw_nj = dw.reshape(_W // _MERGE, _MERGE)[None, :, None, :]
    i0 = (hf_mi * _GRID + wf_nj).reshape(-1)
    i1 = (hf_mi * _GRID + wc_nj).reshape(-1)
    i2 = (hc_mi * _GRID + wf_nj).reshape(-1)
    i3 = (hc_mi * _GRID + wc_nj).reshape(-1)
    w0 = (1 - dh_mi) * (1 - dw_nj)
    w1 = (1 - dh_mi) * dw_nj
    w2 = dh_mi * (1 - dw_nj)
    w3 = dh_mi * dw_nj
    wall = jnp.stack(
        [jnp.broadcast_to(x, (_H // _MERGE, _W // _MERGE, _MERGE, _MERGE))
         for x in (w0, w1, w2, w3)], axis=-1).reshape(-1)

    cvec = jnp.full((16,), (jnp.asarray(num_frames) - _F), dtype=jnp.float32)

    return _sc_interp(pos_embed, i0, i1, i2, i3, wall, cvec)


# speedup vs baseline: 3.1834x; 1.4111x over previous
import functools

import jax
import jax.numpy as jnp
from jax import lax
from jax.experimental import pallas as pl
from jax.experimental.pallas import tpu as pltpu
from jax.experimental.pallas import tpu_sc as plsc

_NUM_POS = 2304
_HIDDEN = 1152
_MERGE = 2
_GRID = 48
_F = 4
_H = 128
_W = 128
_ROWS = _H * _W
_NW = 32
_RPW = _ROWS // _NW
_B = 8
_NCH = _RPW // _B
_NS2 = _NCH // 4
_NG = _HIDDEN // 32


def _linspace(stop, num, num_static):
    div = (jnp.asarray(num) - 1).astype(jnp.float32)
    delta = jnp.float32(stop) / div
    body = lax.iota(jnp.float32, num_static - 1) * delta
    return jnp.concatenate([body, jnp.full((1,), stop, dtype=jnp.float32)])


def _sc_body(table_hbm, i0_hbm, i1_hbm, i2_hbm, i3_hbm,
             wall_hbm, c_hbm, out_hbm,
             i0v, i1v, i2v, i3v, wallv, cv,
             r00, r01, r02, r03, r10, r11, r12, r13,
             ovA, ovB, gs0, gs1, osA, osB):
    wid = lax.axis_index("s") * 2 + lax.axis_index("c")
    base = wid * _RPW
    pltpu.sync_copy(i0_hbm.at[pl.ds(base, _RPW)], i0v)
    pltpu.sync_copy(i1_hbm.at[pl.ds(base, _RPW)], i1v)
    pltpu.sync_copy(i2_hbm.at[pl.ds(base, _RPW)], i2v)
    pltpu.sync_copy(i3_hbm.at[pl.ds(base, _RPW)], i3v)
    pltpu.sync_copy(wall_hbm.at[pl.ds(base * 4, _RPW * 4)],
                    wallv.at[pl.ds(0, _RPW * 4)])
    pltpu.sync_copy(c_hbm, cv)
    idxv = (i0v, i1v, i2v, i3v)
    gbuf = ((r00, r01, r02, r03), (r10, r11, r12, r13))

    lane = lax.iota(jnp.int32, 16)
    sh_e = (2 * lane) & 15
    sh_o = sh_e + 1
    lo_half = lane < 8

    def issue_g(row_off, bufs, sem):
        for k in range(4):
            pltpu.async_copy(
                table_hbm.at[idxv[k].at[pl.ds(row_off, _B)]], bufs[k], sem)

    def wait_g(bufs, sem):
        for k in range(4):
            pltpu.make_async_copy(
                table_hbm.at[pl.ds(0, _B)], bufs[k], sem).wait()

    def issue_w(row_off, ovref, sem):
        for f in range(4):
            pltpu.async_copy(
                ovref, out_hbm.at[pl.ds(f * _ROWS + row_off, 2 * _B)], sem)

    def wait_w(ovref, sem):
        for f in range(4):
            pltpu.make_async_copy(
                ovref, out_hbm.at[pl.ds(0, 2 * _B)], sem).wait()

    def compute(row_off, bufs, ovref, ovbase):
        b0, b1, b2, b3 = bufs
        cvec = cv[...]

        def pos_body(p, _):
            wq = wallv[pl.ds((row_off + p) * 4, 16)]
            a0 = wq[0]
            a1 = wq[1]
            a2 = wq[2]
            a3 = wq[3]
            for g in range(_NG):
                acc0 = (a0 * b0[p, pl.ds(g * 32, 16)]
                        + a1 * b1[p, pl.ds(g * 32, 16)]
                        + a2 * b2[p, pl.ds(g * 32, 16)]
                        + a3 * b3[p, pl.ds(g * 32, 16)] + cvec)
                acc1 = (a0 * b0[p, pl.ds(g * 32 + 16, 16)]
                        + a1 * b1[p, pl.ds(g * 32 + 16, 16)]
                        + a2 * b2[p, pl.ds(g * 32 + 16, 16)]
                        + a3 * b3[p, pl.ds(g * 32 + 16, 16)] + cvec)
                ce = jnp.where(lo_half, jnp.take(acc0, sh_e), jnp.take(acc1, sh_e))
                co = jnp.where(lo_half, jnp.take(acc0, sh_o), jnp.take(acc1, sh_o))
                ovref[ovbase + p, pl.ds(g * 32, 32)] = plsc.pack(
                    ce, co, format=plsc.PackFormat.INTERLEAVED)
            return ()

        lax.fori_loop(0, _B, pos_body, ())

    issue_g(0, gbuf[0], gs0)
    issue_g(_B, gbuf[1], gs1)

    def s2_body(s2, _):
        co = s2 * 4 * _B

        @pl.when(s2 > 0)
        def _():
            wait_w(ovA, osA)
            wait_w(ovB, osB)

        wait_g(gbuf[0], gs0)
        compute(co, gbuf[0], ovA, 0)
        issue_g(co + 2 * _B, gbuf[0], gs0)

        wait_g(gbuf[1], gs1)
        compute(co + _B, gbuf[1], ovA, _B)
        issue_w(base + co, ovA, osA)
        issue_g(co + 3 * _B, gbuf[1], gs1)

        wait_g(gbuf[0], gs0)
        compute(co + 2 * _B, gbuf[0], ovB, 0)

        @pl.when(s2 < _NS2 - 1)
        def _():
            issue_g(co + 4 * _B, gbuf[0], gs0)

        wait_g(gbuf[1], gs1)
        compute(co + 3 * _B, gbuf[1], ovB, _B)
        issue_w(base + co + 2 * _B, ovB, osB)

        @pl.when(s2 < _NS2 - 1)
        def _():
            issue_g(co + 5 * _B, gbuf[1], gs1)

        return ()

    lax.fori_loop(0, _NS2, s2_body, ())
    wait_w(ovA, osA)
    wait_w(ovB, osB)


@functools.partial(
    pl.kernel,
    out_type=jax.ShapeDtypeStruct((_F * _ROWS, _HIDDEN), jnp.bfloat16),
    mesh=plsc.VectorSubcoreMesh(core_axis_name="c", subcore_axis_name="s"),
    compiler_params=pltpu.CompilerParams(needs_layout_passes=False),
    scratch_types=[
        pltpu.VMEM((_RPW,), jnp.int32),
        pltpu.VMEM((_RPW,), jnp.int32),
        pltpu.VMEM((_RPW,), jnp.int32),
        pltpu.VMEM((_RPW,), jnp.int32),
        pltpu.VMEM((_RPW * 4 + 16,), jnp.float32),
        pltpu.VMEM((16,), jnp.float32),
        pltpu.VMEM((_B, _HIDDEN), jnp.float32),
        pltpu.VMEM((_B, _HIDDEN), jnp.float32),
        pltpu.VMEM((_B, _HIDDEN), jnp.float32),
        pltpu.VMEM((_B, _HIDDEN), jnp.float32),
        pltpu.VMEM((_B, _HIDDEN), jnp.float32),
        pltpu.VMEM((_B, _HIDDEN), jnp.float32),
        pltpu.VMEM((_B, _HIDDEN), jnp.float32),
        pltpu.VMEM((_B, _HIDDEN), jnp.float32),
        pltpu.VMEM((2 * _B, _HIDDEN), jnp.bfloat16),
        pltpu.VMEM((2 * _B, _HIDDEN), jnp.bfloat16),
        pltpu.SemaphoreType.DMA,
        pltpu.SemaphoreType.DMA,
        pltpu.SemaphoreType.DMA,
        pltpu.SemaphoreType.DMA,
    ],
)
def _sc_interp(*refs):
    _sc_body(*refs)


def kernel(num_frames, height, width, pos_embed):
    h_idxs = _linspace(_GRID - 1, height, _H)
    w_idxs = _linspace(_GRID - 1, width, _W)
    hf = jnp.floor(h_idxs).astype(jnp.int32)
    wf = jnp.floor(w_idxs).astype(jnp.int32)
    hc = jnp.minimum(hf + 1, _GRID - 1)
    wc = jnp.minimum(wf + 1, _GRID - 1)
    dh = h_idxs - hf
    dw = w_idxs - wf

    hf_mi = hf.reshape(_H // _MERGE, _MERGE)[:, None, :, None]
    hc_mi = hc.reshape(_H // _MERGE, _MERGE)[:, None, :, None]
    dh_mi = dh.reshape(_H // _MERGE, _MERGE)[:, None, :, None]
    wf_nj = wf.reshape(_W // _MERGE, _MERGE)[None, :, None, :]
    wc_nj = wc.reshape(_W // _MERGE, _MERGE)[None, :, None, :]
    dw_nj = dw.reshape(_W // _MERGE, _MERGE)[None, :, None, :]
    i0 = (hf_mi * _GRID + wf_nj).reshape(-1)
    i1 = (hf_mi * _GRID + wc_nj).reshape(-1)
    i2 = (hc_mi * _GRID + wf_nj).reshape(-1)
    i3 = (hc_mi * _GRID + wc_nj).reshape(-1)
    w0 = (1 - dh_mi) * (1 - dw_nj)
    w1 = (1 - dh_mi) * dw_nj
    w2 = dh_mi * (1 - dw_nj)
    w3 = dh_mi * dw_nj
    wall = jnp.stack(
        [jnp.broadcast_to(x, (_H // _MERGE, _W // _MERGE, _MERGE, _MERGE))
         for x in (w0, w1, w2, w3)], axis=-1).reshape(-1)

    cvec = jnp.full((16,), (jnp.asarray(num_frames) - _F), dtype=jnp.float32)

    return _sc_interp(pos_embed, i0, i1, i2, i3, wall, cvec)
